# baseline (speedup 1.0000x reference)
import functools

import jax
import jax.numpy as jnp
from jax import lax
from jax.experimental import pallas as pl
from jax.experimental.pallas import tpu as pltpu
from jax.experimental.pallas import tpu_sc as plsc

NC = 2
NS = 16
NW = NC * NS

SUB = 128
CHUNK = 512
SUBS = CHUNK // SUB


def _emb_body(total_rows, d_model, txt_hbm, seg_hbm, pos_hbm, tok_tab_hbm,
              pos_tab_hbm, seg_tab_hbm, out_hbm, tok_idx, seg_idx, pos_idx,
              rows, sem, sem_add):
  wid = lax.axis_index("s") * NC + lax.axis_index("c")
  per_w = total_rows // NW
  n_chunks = per_w // CHUNK
  base = wid * per_w

  def chunk_body(c, carry):
    off = base + c * CHUNK
    for j in range(SUBS):
      pltpu.sync_copy(txt_hbm.at[pl.ds(off + j * SUB, SUB)], tok_idx.at[j])
      pltpu.sync_copy(seg_hbm.at[pl.ds(off + j * SUB, SUB)], seg_idx.at[j])
      pltpu.sync_copy(pos_hbm.at[pl.ds(off + j * SUB, SUB)], pos_idx.at[j])
    for j in range(SUBS):
      pltpu.async_copy(tok_tab_hbm.at[tok_idx.at[j]],
                       rows.at[pl.ds(j * SUB, SUB)], sem)
    for j in range(SUBS):
      pltpu.make_async_copy(tok_tab_hbm.at[tok_idx.at[j]],
                            rows.at[pl.ds(j * SUB, SUB)], sem).wait()
    for j in range(SUBS):
      pltpu.async_copy(pos_tab_hbm.at[pos_idx.at[j]],
                       rows.at[pl.ds(j * SUB, SUB)], sem_add, add=True)
      pltpu.async_copy(seg_tab_hbm.at[seg_idx.at[j]],
                       rows.at[pl.ds(j * SUB, SUB)], sem_add, add=True)
    for j in range(SUBS):
      pltpu.make_async_copy(pos_tab_hbm.at[pos_idx.at[j]],
                            rows.at[pl.ds(j * SUB, SUB)], sem_add).wait()
      pltpu.make_async_copy(seg_tab_hbm.at[seg_idx.at[j]],
                            rows.at[pl.ds(j * SUB, SUB)], sem_add).wait()
    pltpu.sync_copy(rows, out_hbm.at[pl.ds(off, CHUNK)])
    return carry

  lax.fori_loop(0, n_chunks, chunk_body, 0)


def kernel(txt, seg, tok_table, pos_table, seg_table):
  B, L = txt.shape
  D = tok_table.shape[1]
  N = B * L

  txt_flat = txt.reshape(N).astype(jnp.int32)
  seg_flat = seg.reshape(N).astype(jnp.int32)
  pos_flat = jnp.broadcast_to(
      jnp.arange(L, dtype=jnp.int32)[None, :], (B, L)).reshape(N)

  mesh = plsc.VectorSubcoreMesh(core_axis_name="c", subcore_axis_name="s")
  k = pl.kernel(
      functools.partial(_emb_body, N, D),
      out_type=jax.ShapeDtypeStruct((N, D), jnp.float32),
      mesh=mesh,
      compiler_params=pltpu.CompilerParams(use_tc_tiling_on_sc=False),
      scratch_types=[
          pltpu.VMEM((SUBS, SUB), jnp.int32),
          pltpu.VMEM((SUBS, SUB), jnp.int32),
          pltpu.VMEM((SUBS, SUB), jnp.int32),
          pltpu.VMEM((CHUNK, D), jnp.float32),
          pltpu.SemaphoreType.DMA,
          pltpu.SemaphoreType.DMA,
      ],
  )
  out = k(txt_flat, seg_flat, pos_flat, tok_table, pos_table, seg_table)
  return out.reshape(B, L, D)


# speedup vs baseline: 9.3326x; 9.3326x over previous
import functools

import jax
import jax.numpy as jnp
from jax import lax
from jax.experimental import pallas as pl
from jax.experimental.pallas import tpu as pltpu
from jax.experimental.pallas import tpu_sc as plsc

NC = 2
NS = 16
NW = NC * NS

SUB = 128
CHUNK = 512
SUBS = CHUNK // SUB
L16 = 16


def _emb_body(total_rows, seq_len, txt_hbm, seg_hbm, tok_tab_hbm, psg_hbm,
              out_hbm, psg_sh, tok_idx, seg_raw, fidx, rows,
              sem_g0, sem_g1, sem_w0, sem_w1, sem_a):
  wid = lax.axis_index("s") * NC + lax.axis_index("c")
  per_w = total_rows // NW
  n_chunks = per_w // CHUNK
  base = wid * per_w
  iota = lax.iota(jnp.int32, L16)
  sem_g = (sem_g0, sem_g1)
  sem_w = (sem_w0, sem_w1)

  @pl.when(lax.axis_index("s") == 0)
  def _():
    pltpu.sync_copy(psg_hbm, psg_sh)
  plsc.subcore_barrier()

  def _stage_and_fuse(cc, b):
    off = base + cc * CHUNK
    pltpu.sync_copy(txt_hbm.at[pl.ds(off, CHUNK)], tok_idx.at[b])
    pltpu.sync_copy(seg_hbm.at[pl.ds(off, CHUNK)], seg_raw.at[b])
    for g in range(CHUNK // L16):
      pos_v = lax.rem(off + (g * L16) + iota, seq_len)
      seg_v = seg_raw[b, pl.ds(g * L16, L16)]
      fidx[b, pl.ds(g * L16, L16)] = pos_v * 3 + seg_v

  def _start_gather(b):
    for j in range(SUBS):
      pltpu.async_copy(tok_tab_hbm.at[tok_idx.at[b, pl.ds(j * SUB, SUB)]],
                       rows.at[b, pl.ds(j * SUB, SUB)], sem_g[b])

  def _wait_gather(b):
    for j in range(SUBS):
      pltpu.make_async_copy(tok_tab_hbm.at[tok_idx.at[b, pl.ds(j * SUB, SUB)]],
                            rows.at[b, pl.ds(j * SUB, SUB)], sem_g[b]).wait()

  def _write_desc(cc, b):
    off = base + cc * CHUNK
    return pltpu.make_async_copy(rows.at[b], out_hbm.at[pl.ds(off, CHUNK)],
                                 sem_w[b])

  _stage_and_fuse(0, 0)
  _start_gather(0)

  @pl.loop(0, n_chunks, step=2)
  def _chunks(c):
    for b in range(2):
      cc = c + b
      nb = 1 - b

      @pl.when(cc + 1 < n_chunks)
      def _():
        _stage_and_fuse(cc + 1, nb)

        @pl.when(cc >= 1)
        def _():
          _write_desc(cc - 1, nb).wait()
        _start_gather(nb)

      _wait_gather(b)
      for j in range(SUBS):
        pltpu.async_copy(psg_sh.at[fidx.at[b, pl.ds(j * SUB, SUB)]],
                         rows.at[b, pl.ds(j * SUB, SUB)], sem_a, add=True)
      for j in range(SUBS):
        pltpu.make_async_copy(psg_sh.at[fidx.at[b, pl.ds(j * SUB, SUB)]],
                              rows.at[b, pl.ds(j * SUB, SUB)], sem_a).wait()
      _write_desc(cc, b).start()

  _write_desc(n_chunks - 2, 0).wait()
  _write_desc(n_chunks - 1, 1).wait()


def kernel(txt, seg, tok_table, pos_table, seg_table):
  B, L = txt.shape
  D = tok_table.shape[1]
  N = B * L

  txt_flat = txt.reshape(N).astype(jnp.int32)
  seg_flat = seg.reshape(N).astype(jnp.int32)
  psg = (pos_table[:, None, :] + seg_table[None, :, :]).reshape(L * 3, D)

  mesh = plsc.VectorSubcoreMesh(core_axis_name="c", subcore_axis_name="s")
  k = pl.kernel(
      functools.partial(_emb_body, N, L),
      out_type=jax.ShapeDtypeStruct((N, D), jnp.float32),
      mesh=mesh,
      compiler_params=pltpu.CompilerParams(use_tc_tiling_on_sc=False),
      scratch_types=[
          pltpu.VMEM_SHARED((L * 3, D), jnp.float32),
          pltpu.VMEM((2, CHUNK), jnp.int32),
          pltpu.VMEM((2, CHUNK), jnp.int32),
          pltpu.VMEM((2, CHUNK), jnp.int32),
          pltpu.VMEM((2, CHUNK, D), jnp.float32),
          pltpu.SemaphoreType.DMA,
          pltpu.SemaphoreType.DMA,
          pltpu.SemaphoreType.DMA,
          pltpu.SemaphoreType.DMA,
          pltpu.SemaphoreType.DMA,
      ],
  )
  out = k(txt_flat, seg_flat, tok_table, psg)
  return out.reshape(B, L, D)


# speedup vs baseline: 9.5211x; 1.0202x over previous
import functools

import jax
import jax.numpy as jnp
from jax import lax
from jax.experimental import pallas as pl
from jax.experimental.pallas import tpu as pltpu
from jax.experimental.pallas import tpu_sc as plsc

NC = 2
NS = 16
NW = NC * NS

SUB = 128
CHUNK = 256
SUBS = CHUNK // SUB
NBUF = 4
L16 = 16


def _emb_body(total_rows, seq_len, txt_hbm, seg_hbm, tok_tab_hbm, psg_hbm,
              out_hbm, psg_sh, tok_idx, seg_raw, fidx, rows,
              sem_g0, sem_g1, sem_g2, sem_g3,
              sem_w0, sem_w1, sem_w2, sem_w3,
              sem_i0, sem_i1, sem_i2, sem_i3, sem_a):
  wid = lax.axis_index("s") * NC + lax.axis_index("c")
  per_w = total_rows // NW
  n_chunks = per_w // CHUNK
  base = wid * per_w
  iota = lax.iota(jnp.int32, L16)
  sem_g = (sem_g0, sem_g1, sem_g2, sem_g3)
  sem_w = (sem_w0, sem_w1, sem_w2, sem_w3)
  sem_i = (sem_i0, sem_i1, sem_i2, sem_i3)

  @pl.when(lax.axis_index("s") == 0)
  def _():
    pltpu.sync_copy(psg_hbm, psg_sh)
  plsc.subcore_barrier()

  def _stage_descs(cc, b):
    off = base + cc * CHUNK
    return (
        pltpu.make_async_copy(txt_hbm.at[pl.ds(off, CHUNK)], tok_idx.at[b],
                              sem_i[b]),
        pltpu.make_async_copy(seg_hbm.at[pl.ds(off, CHUNK)], seg_raw.at[b],
                              sem_i[b]),
    )

  def _start_stage(cc, b):
    for d in _stage_descs(cc, b):
      d.start()

  def _finish_stage(cc, b):
    for d in _stage_descs(cc, b):
      d.wait()
    off = base + cc * CHUNK
    for g in range(CHUNK // L16):
      pos_v = lax.rem(off + (g * L16) + iota, seq_len)
      seg_v = seg_raw[b, pl.ds(g * L16, L16)]
      fidx[b, pl.ds(g * L16, L16)] = pos_v * 3 + seg_v

  def _gather_descs(b):
    return [
        pltpu.make_async_copy(
            tok_tab_hbm.at[tok_idx.at[b, pl.ds(j * SUB, SUB)]],
            rows.at[b, pl.ds(j * SUB, SUB)], sem_g[b])
        for j in range(SUBS)
    ]

  def _write_desc(cc, b):
    off = base + cc * CHUNK
    return pltpu.make_async_copy(rows.at[b], out_hbm.at[pl.ds(off, CHUNK)],
                                 sem_w[b])

  _start_stage(0, 0)
  _start_stage(1, 1)
  _finish_stage(0, 0)
  for d in _gather_descs(0):
    d.start()

  @pl.loop(0, n_chunks, step=NBUF)
  def _chunks(c):
    for u in range(NBUF):
      cc = c + u
      b = u
      b1 = (u + 1) % NBUF
      b2 = (u + 2) % NBUF

      @pl.when(cc + 2 < n_chunks)
      def _():
        _start_stage(cc + 2, b2)

      @pl.when(cc + 1 < n_chunks)
      def _():
        _finish_stage(cc + 1, b1)

        @pl.when(cc + 1 >= NBUF)
        def _():
          _write_desc(cc + 1 - NBUF, b1).wait()
        for d in _gather_descs(b1):
          d.start()

      for d in _gather_descs(b):
        d.wait()
      for j in range(SUBS):
        pltpu.async_copy(psg_sh.at[fidx.at[b, pl.ds(j * SUB, SUB)]],
                         rows.at[b, pl.ds(j * SUB, SUB)], sem_a, add=True)
      for j in range(SUBS):
        pltpu.make_async_copy(psg_sh.at[fidx.at[b, pl.ds(j * SUB, SUB)]],
                              rows.at[b, pl.ds(j * SUB, SUB)], sem_a).wait()
      _write_desc(cc, b).start()

  for u in range(NBUF):
    cc = n_chunks - NBUF + u
    _write_desc(cc, cc % NBUF).wait()


def kernel(txt, seg, tok_table, pos_table, seg_table):
  B, L = txt.shape
  D = tok_table.shape[1]
  N = B * L

  txt_flat = txt.reshape(N).astype(jnp.int32)
  seg_flat = seg.reshape(N).astype(jnp.int32)
  psg = (pos_table[:, None, :] + seg_table[None, :, :]).reshape(L * 3, D)

  mesh = plsc.VectorSubcoreMesh(core_axis_name="c", subcore_axis_name="s")
  k = pl.kernel(
      functools.partial(_emb_body, N, L),
      out_type=jax.ShapeDtypeStruct((N, D), jnp.float32),
      mesh=mesh,
      compiler_params=pltpu.CompilerParams(use_tc_tiling_on_sc=False),
      scratch_types=[
          pltpu.VMEM_SHARED((L * 3, D), jnp.float32),
          pltpu.VMEM((NBUF, CHUNK), jnp.int32),
          pltpu.VMEM((NBUF, CHUNK), jnp.int32),
          pltpu.VMEM((NBUF, CHUNK), jnp.int32),
          pltpu.VMEM((NBUF, CHUNK, D), jnp.float32),
          pltpu.SemaphoreType.DMA,
          pltpu.SemaphoreType.DMA,
          pltpu.SemaphoreType.DMA,
          pltpu.SemaphoreType.DMA,
          pltpu.SemaphoreType.DMA,
          pltpu.SemaphoreType.DMA,
          pltpu.SemaphoreType.DMA,
          pltpu.SemaphoreType.DMA,
          pltpu.SemaphoreType.DMA,
          pltpu.SemaphoreType.DMA,
          pltpu.SemaphoreType.DMA,
          pltpu.SemaphoreType.DMA,
          pltpu.SemaphoreType.DMA,
      ],
  )
  out = k(txt_flat, seg_flat, tok_table, psg)
  return out.reshape(B, L, D)


# speedup vs baseline: 9.5230x; 1.0002x over previous
import functools

import jax
import jax.numpy as jnp
from jax import lax
from jax.experimental import pallas as pl
from jax.experimental.pallas import tpu as pltpu
from jax.experimental.pallas import tpu_sc as plsc

NC = 2
NS = 16
NW = NC * NS

SUB = 256
CHUNK = 256
SUBS = CHUNK // SUB
NBUF = 4
L16 = 16


def _emb_body(total_rows, seq_len, txt_hbm, seg_hbm, tok_tab_hbm, psg_hbm,
              out_hbm, psg_sh, tok_idx, seg_raw, fidx, rows,
              sem_g0, sem_g1, sem_g2, sem_g3,
              sem_w0, sem_w1, sem_w2, sem_w3,
              sem_i0, sem_i1, sem_i2, sem_i3, sem_a):
  wid = lax.axis_index("s") * NC + lax.axis_index("c")
  per_w = total_rows // NW
  n_chunks = per_w // CHUNK
  base = wid * per_w
  iota = lax.iota(jnp.int32, L16)
  sem_g = (sem_g0, sem_g1, sem_g2, sem_g3)
  sem_w = (sem_w0, sem_w1, sem_w2, sem_w3)
  sem_i = (sem_i0, sem_i1, sem_i2, sem_i3)

  @pl.when(lax.axis_index("s") == 0)
  def _():
    pltpu.sync_copy(psg_hbm, psg_sh)
  plsc.subcore_barrier()

  def _stage_descs(cc, b):
    off = base + cc * CHUNK
    return (
        pltpu.make_async_copy(txt_hbm.at[pl.ds(off, CHUNK)], tok_idx.at[b],
                              sem_i[b]),
        pltpu.make_async_copy(seg_hbm.at[pl.ds(off, CHUNK)], seg_raw.at[b],
                              sem_i[b]),
    )

  def _start_stage(cc, b):
    for d in _stage_descs(cc, b):
      d.start()

  def _finish_stage(cc, b):
    for d in _stage_descs(cc, b):
      d.wait()
    off = base + cc * CHUNK
    for g in range(CHUNK // L16):
      pos_v = lax.rem(off + (g * L16) + iota, seq_len)
      seg_v = seg_raw[b, pl.ds(g * L16, L16)]
      fidx[b, pl.ds(g * L16, L16)] = pos_v * 3 + seg_v

  def _gather_descs(b):
    return [
        pltpu.make_async_copy(
            tok_tab_hbm.at[tok_idx.at[b, pl.ds(j * SUB, SUB)]],
            rows.at[b, pl.ds(j * SUB, SUB)], sem_g[b])
        for j in range(SUBS)
    ]

  def _write_desc(cc, b):
    off = base + cc * CHUNK
    return pltpu.make_async_copy(rows.at[b], out_hbm.at[pl.ds(off, CHUNK)],
                                 sem_w[b])

  _start_stage(0, 0)
  _start_stage(1, 1)
  _finish_stage(0, 0)
  for d in _gather_descs(0):
    d.start()

  @pl.loop(0, n_chunks, step=NBUF)
  def _chunks(c):
    for u in range(NBUF):
      cc = c + u
      b = u
      b1 = (u + 1) % NBUF
      b2 = (u + 2) % NBUF

      @pl.when(cc + 2 < n_chunks)
      def _():
        _start_stage(cc + 2, b2)

      @pl.when(cc + 1 < n_chunks)
      def _():
        _finish_stage(cc + 1, b1)

        @pl.when(cc + 1 >= NBUF)
        def _():
          _write_desc(cc + 1 - NBUF, b1).wait()
        for d in _gather_descs(b1):
          d.start()

      for d in _gather_descs(b):
        d.wait()
      for j in range(SUBS):
        pltpu.async_copy(psg_sh.at[fidx.at[b, pl.ds(j * SUB, SUB)]],
                         rows.at[b, pl.ds(j * SUB, SUB)], sem_a, add=True)
      for j in range(SUBS):
        pltpu.make_async_copy(psg_sh.at[fidx.at[b, pl.ds(j * SUB, SUB)]],
                              rows.at[b, pl.ds(j * SUB, SUB)], sem_a).wait()
      _write_desc(cc, b).start()

  for u in range(NBUF):
    cc = n_chunks - NBUF + u
    _write_desc(cc, cc % NBUF).wait()


def kernel(txt, seg, tok_table, pos_table, seg_table):
  B, L = txt.shape
  D = tok_table.shape[1]
  N = B * L

  txt_flat = txt.reshape(N).astype(jnp.int32)
  seg_flat = seg.reshape(N).astype(jnp.int32)
  psg = (pos_table[:, None, :] + seg_table[None, :, :]).reshape(L * 3, D)

  mesh = plsc.VectorSubcoreMesh(core_axis_name="c", subcore_axis_name="s")
  k = pl.kernel(
      functools.partial(_emb_body, N, L),
      out_type=jax.ShapeDtypeStruct((N, D), jnp.float32),
      mesh=mesh,
      compiler_params=pltpu.CompilerParams(use_tc_tiling_on_sc=False),
      scratch_types=[
          pltpu.VMEM_SHARED((L * 3, D), jnp.float32),
          pltpu.VMEM((NBUF, CHUNK), jnp.int32),
          pltpu.VMEM((NBUF, CHUNK), jnp.int32),
          pltpu.VMEM((NBUF, CHUNK), jnp.int32),
          pltpu.VMEM((NBUF, CHUNK, D), jnp.float32),
          pltpu.SemaphoreType.DMA,
          pltpu.SemaphoreType.DMA,
          pltpu.SemaphoreType.DMA,
          pltpu.SemaphoreType.DMA,
          pltpu.SemaphoreType.DMA,
          pltpu.SemaphoreType.DMA,
          pltpu.SemaphoreType.DMA,
          pltpu.SemaphoreType.DMA,
          pltpu.SemaphoreType.DMA,
          pltpu.SemaphoreType.DMA,
          pltpu.SemaphoreType.DMA,
          pltpu.SemaphoreType.DMA,
          pltpu.SemaphoreType.DMA,
      ],
  )
  out = k(txt_flat, seg_flat, tok_table, psg)
  return out.reshape(B, L, D)


# speedup vs baseline: 9.5333x; 1.0011x over previous
import functools

import jax
import jax.numpy as jnp
from jax import lax
from jax.experimental import pallas as pl
from jax.experimental.pallas import tpu as pltpu
from jax.experimental.pallas import tpu_sc as plsc

NC = 2
NS = 16
NW = NC * NS

SUB = 256
CHUNK = 256
SUBS = CHUNK // SUB
NBUF = 4
L16 = 16


def _emb_body(total_rows, seq_len, txt_hbm, seg_hbm, tok_tab_hbm, psg_hbm,
              out_hbm, psg_sh, tok_idx, seg_raw, fidx, rows,
              sem_g0, sem_g1, sem_g2, sem_g3,
              sem_w0, sem_w1, sem_w2, sem_w3,
              sem_i0, sem_i1, sem_i2, sem_i3, sem_a):
  wid = lax.axis_index("s") * NC + lax.axis_index("c")
  per_w = total_rows // NW
  n_chunks = per_w // CHUNK
  base = wid * per_w
  iota = lax.iota(jnp.int32, L16)
  sem_g = (sem_g0, sem_g1, sem_g2, sem_g3)
  sem_w = (sem_w0, sem_w1, sem_w2, sem_w3)
  sem_i = (sem_i0, sem_i1, sem_i2, sem_i3)

  @pl.when(lax.axis_index("s") == 0)
  def _():
    pltpu.sync_copy(psg_hbm, psg_sh)
  plsc.subcore_barrier()

  def _stage_descs(cc, b):
    off = base + cc * CHUNK
    return (
        pltpu.make_async_copy(txt_hbm.at[pl.ds(off, CHUNK)], tok_idx.at[b],
                              sem_i[b]),
        pltpu.make_async_copy(seg_hbm.at[pl.ds(off, CHUNK)], seg_raw.at[b],
                              sem_i[b]),
    )

  def _start_stage(cc, b):
    for d in _stage_descs(cc, b):
      d.start()

  def _finish_stage(cc, b):
    for d in _stage_descs(cc, b):
      d.wait()
    off = base + cc * CHUNK
    for g in range(CHUNK // L16):
      pos_v = lax.rem(off + (g * L16) + iota, seq_len)
      seg_v = seg_raw[b, pl.ds(g * L16, L16)]
      fidx[b, pl.ds(g * L16, L16)] = pos_v * 3 + seg_v

  def _start_gathers(b):
    for g in range(CHUNK // L16):
      idx_vals = tok_idx[b, pl.ds(g * L16, L16)]
      pltpu.async_copy(tok_tab_hbm.at[idx_vals],
                       rows.at[b, pl.ds(g * L16, L16)], sem_g[b])

  def _gather_descs(b):
    return [
        pltpu.make_async_copy(
            tok_tab_hbm.at[tok_idx.at[b, pl.ds(j * SUB, SUB)]],
            rows.at[b, pl.ds(j * SUB, SUB)], sem_g[b])
        for j in range(SUBS)
    ]

  def _write_desc(cc, b):
    off = base + cc * CHUNK
    return pltpu.make_async_copy(rows.at[b], out_hbm.at[pl.ds(off, CHUNK)],
                                 sem_w[b])

  _start_stage(0, 0)
  _start_stage(1, 1)
  _finish_stage(0, 0)
  _start_gathers(0)

  @pl.loop(0, n_chunks, step=NBUF)
  def _chunks(c):
    for u in range(NBUF):
      cc = c + u
      b = u
      b1 = (u + 1) % NBUF
      b2 = (u + 2) % NBUF

      @pl.when(cc + 2 < n_chunks)
      def _():
        _start_stage(cc + 2, b2)

      @pl.when(cc + 1 < n_chunks)
      def _():
        _finish_stage(cc + 1, b1)

        @pl.when(cc + 1 >= NBUF)
        def _():
          _write_desc(cc + 1 - NBUF, b1).wait()
        _start_gathers(b1)

      for d in _gather_descs(b):
        d.wait()
      for j in range(SUBS):
        pltpu.async_copy(psg_sh.at[fidx.at[b, pl.ds(j * SUB, SUB)]],
                         rows.at[b, pl.ds(j * SUB, SUB)], sem_a, add=True)
      for j in range(SUBS):
        pltpu.make_async_copy(psg_sh.at[fidx.at[b, pl.ds(j * SUB, SUB)]],
                              rows.at[b, pl.ds(j * SUB, SUB)], sem_a).wait()
      _write_desc(cc, b).start()

  for u in range(NBUF):
    cc = n_chunks - NBUF + u
    _write_desc(cc, cc % NBUF).wait()


def kernel(txt, seg, tok_table, pos_table, seg_table):
  B, L = txt.shape
  D = tok_table.shape[1]
  N = B * L

  txt_flat = txt.reshape(N).astype(jnp.int32)
  seg_flat = seg.reshape(N).astype(jnp.int32)
  psg = (pos_table[:, None, :] + seg_table[None, :, :]).reshape(L * 3, D)

  mesh = plsc.VectorSubcoreMesh(core_axis_name="c", subcore_axis_name="s")
  k = pl.kernel(
      functools.partial(_emb_body, N, L),
      out_type=jax.ShapeDtypeStruct((N, D), jnp.float32),
      mesh=mesh,
      compiler_params=pltpu.CompilerParams(use_tc_tiling_on_sc=False),
      scratch_types=[
          pltpu.VMEM_SHARED((L * 3, D), jnp.float32),
          pltpu.VMEM((NBUF, CHUNK), jnp.int32),
          pltpu.VMEM((NBUF, CHUNK), jnp.int32),
          pltpu.VMEM((NBUF, CHUNK), jnp.int32),
          pltpu.VMEM((NBUF, CHUNK, D), jnp.float32),
          pltpu.SemaphoreType.DMA,
          pltpu.SemaphoreType.DMA,
          pltpu.SemaphoreType.DMA,
          pltpu.SemaphoreType.DMA,
          pltpu.SemaphoreType.DMA,
          pltpu.SemaphoreType.DMA,
          pltpu.SemaphoreType.DMA,
          pltpu.SemaphoreType.DMA,
          pltpu.SemaphoreType.DMA,
          pltpu.SemaphoreType.DMA,
          pltpu.SemaphoreType.DMA,
          pltpu.SemaphoreType.DMA,
          pltpu.SemaphoreType.DMA,
      ],
  )
  out = k(txt_flat, seg_flat, tok_table, psg)
  return out.reshape(B, L, D)
